# baseline (speedup 1.0000x reference)
import functools
import math

import jax
import jax.numpy as jnp
from jax import lax
from jax.experimental import pallas as pl
from jax.experimental.pallas import tpu as pltpu
from jax.experimental.pallas import tpu_sc as plsc

D = 64
LANES = 16
CH = 256
IDX_BLK = 128
NC = 2
NS = 16
NW = NC * NS
S = 512


def _sc_body(scale, n_rows, tok_hbm, x_hbm, seg_hbm, c_hbm, out_hbm,
             xidx_all, cidx_all, toka, cba, tokb, cbb,
             gsem_a, gsem_b, ssem_a, ssem_b):
    wid = lax.axis_index("s") * NC + lax.axis_index("c")
    rows_per_w = n_rows // NW
    n_chunks = rows_per_w // CH
    n_half = n_chunks // 2
    idx_rows = rows_per_w // IDX_BLK
    iota = lax.iota(jnp.int32, LANES)

    pltpu.sync_copy(x_hbm.at[pl.ds(wid * idx_rows, idx_rows)], xidx_all)
    pltpu.sync_copy(seg_hbm.at[pl.ds(wid * idx_rows, idx_rows)], cidx_all)

    @pl.loop(0, idx_rows)
    def _cidx(i):
        pos_base = lax.rem(i, S // IDX_BLK) * IDX_BLK
        for j in range(IDX_BLK // LANES):
            sl = pl.ds(j * LANES, LANES)
            cidx_all[i, sl] = (cidx_all[i, sl] * S
                               + (pos_base + j * LANES + iota))

    def start_gather(c, tokbuf, cbuf, gsem):
        for j in range(CH // IDX_BLK):
            sl = pl.ds(j * IDX_BLK, IDX_BLK)
            pltpu.make_async_copy(
                tok_hbm.at[xidx_all.at[2 * c + j]], tokbuf.at[sl], gsem
            ).start()
            pltpu.make_async_copy(
                c_hbm.at[cidx_all.at[2 * c + j]], cbuf.at[sl], gsem
            ).start()

    def wait_gather(tokbuf, cbuf, gsem):
        pltpu.make_async_copy(tok_hbm.at[pl.ds(0, CH)], tokbuf, gsem).wait()
        pltpu.make_async_copy(c_hbm.at[pl.ds(0, CH)], cbuf, gsem).wait()

    def fma(tokbuf, cbuf):
        @pl.loop(0, CH, unroll=4)
        def _row(r):
            for k in range(D // LANES):
                sl = pl.ds(k * LANES, LANES)
                tokbuf[r, sl] = tokbuf[r, sl] * scale + cbuf[r, sl]

    def out_slice(c):
        return out_hbm.at[pl.ds(wid * rows_per_w + c * CH, CH)]

    start_gather(0, toka, cba, gsem_a)
    start_gather(1, tokb, cbb, gsem_b)

    @pl.loop(0, n_half)
    def _pipe(t):
        c0 = 2 * t
        wait_gather(toka, cba, gsem_a)
        fma(toka, cba)
        pltpu.make_async_copy(toka, out_slice(c0), ssem_a).start()
        wait_gather(tokb, cbb, gsem_b)
        fma(tokb, cbb)
        pltpu.make_async_copy(tokb, out_slice(c0 + 1), ssem_b).start()
        pltpu.make_async_copy(toka, out_slice(c0), ssem_a).wait()
        pltpu.make_async_copy(tokb, out_slice(c0 + 1), ssem_b).wait()

        @pl.when(t + 1 < n_half)
        def _next():
            start_gather(c0 + 2, toka, cba, gsem_a)
            start_gather(c0 + 3, tokb, cbb, gsem_b)


@functools.partial(jax.jit, static_argnames=("n_rows",))
def _sc_embed(token_table, x_idx, seg_idx, comb, n_rows):
    scale = float(math.sqrt(D))
    mesh = plsc.VectorSubcoreMesh(core_axis_name="c", subcore_axis_name="s")
    idx_rows = n_rows // NW // IDX_BLK
    grid_kernel = pl.kernel(
        functools.partial(_sc_body, scale, n_rows),
        out_type=jax.ShapeDtypeStruct((n_rows, D), jnp.float32),
        mesh=mesh,
        compiler_params=pltpu.CompilerParams(use_tc_tiling_on_sc=False),
        scratch_types=[
            pltpu.VMEM((idx_rows, IDX_BLK), jnp.int32),
            pltpu.VMEM((idx_rows, IDX_BLK), jnp.int32),
            pltpu.VMEM((CH, D), jnp.float32),
            pltpu.VMEM((CH, D), jnp.float32),
            pltpu.VMEM((CH, D), jnp.float32),
            pltpu.VMEM((CH, D), jnp.float32),
            pltpu.SemaphoreType.DMA,
            pltpu.SemaphoreType.DMA,
            pltpu.SemaphoreType.DMA,
            pltpu.SemaphoreType.DMA,
        ],
    )
    return grid_kernel(token_table, x_idx, seg_idx, comb)


def kernel(x, segment_info, token_table, pos_embedding, segment_table):
    B, S_in = x.shape
    n_rows = B * S_in
    assert S_in == S and n_rows % (NW * CH) == 0 and CH % IDX_BLK == 0
    x_idx = x.reshape(n_rows // IDX_BLK, IDX_BLK).astype(jnp.int32)
    seg_idx = segment_info.reshape(n_rows // IDX_BLK, IDX_BLK).astype(jnp.int32)
    comb = (pos_embedding[0, :S_in, :][None, :, :]
            + segment_table[:, None, :]).reshape(-1, D)
    out = _sc_embed(token_table, x_idx, seg_idx, comb, n_rows)
    return out.reshape(B, S_in, D)


# speedup vs baseline: 1.1213x; 1.1213x over previous
import functools
import math

import jax
import jax.numpy as jnp
from jax import lax
from jax.experimental import pallas as pl
from jax.experimental.pallas import tpu as pltpu
from jax.experimental.pallas import tpu_sc as plsc

D = 64
LANES = 16
CH = 256
IDX_BLK = 128
NC = 2
NS = 16
NW = NC * NS
S = 512


def _sc_body(scale, n_rows, tok_hbm, x_hbm, seg_hbm, c_hbm, out_hbm,
             xidx_all, cidx_all, toka, cba, tokb, cbb,
             gsem_a, gsem_b, ssem_a, ssem_b):
    wid = lax.axis_index("s") * NC + lax.axis_index("c")
    rows_per_w = n_rows // NW
    n_chunks = rows_per_w // CH
    n_half = n_chunks // 2
    idx_rows = rows_per_w // IDX_BLK
    iota = lax.iota(jnp.int32, LANES)

    pltpu.sync_copy(x_hbm.at[pl.ds(wid * idx_rows, idx_rows)], xidx_all)
    pltpu.sync_copy(seg_hbm.at[pl.ds(wid * idx_rows, idx_rows)], cidx_all)

    @pl.loop(0, idx_rows)
    def _cidx(i):
        pos_base = lax.rem(i, S // IDX_BLK) * IDX_BLK
        for j in range(IDX_BLK // LANES):
            sl = pl.ds(j * LANES, LANES)
            cidx_all[i, sl] = (cidx_all[i, sl] * S
                               + (pos_base + j * LANES + iota))

    def start_gather(c, tokbuf, cbuf, gsem):
        for j in range(CH // IDX_BLK):
            sl = pl.ds(j * IDX_BLK, IDX_BLK)
            pltpu.make_async_copy(
                tok_hbm.at[xidx_all.at[2 * c + j]], tokbuf.at[sl], gsem
            ).start()
            pltpu.make_async_copy(
                c_hbm.at[cidx_all.at[2 * c + j]], cbuf.at[sl], gsem
            ).start()

    def wait_gather(tokbuf, cbuf, gsem):
        pltpu.make_async_copy(tok_hbm.at[pl.ds(0, CH)], tokbuf, gsem).wait()
        pltpu.make_async_copy(c_hbm.at[pl.ds(0, CH)], cbuf, gsem).wait()

    def fma(tokbuf, cbuf):
        @pl.loop(0, CH, unroll=4)
        def _row(r):
            for k in range(D // LANES):
                sl = pl.ds(k * LANES, LANES)
                tokbuf[r, sl] = tokbuf[r, sl] * scale + cbuf[r, sl]

    def out_slice(c):
        return out_hbm.at[pl.ds(wid * rows_per_w + c * CH, CH)]

    start_gather(0, toka, cba, gsem_a)

    @pl.loop(0, n_half)
    def _pipe(t):
        c0 = 2 * t

        @pl.when(t > 0)
        def _free_b():
            pltpu.make_async_copy(tokb, out_slice(c0 - 1), ssem_b).wait()

        start_gather(c0 + 1, tokb, cbb, gsem_b)

        wait_gather(toka, cba, gsem_a)
        fma(toka, cba)
        pltpu.make_async_copy(toka, out_slice(c0), ssem_a).start()

        @pl.when(t + 1 < n_half)
        def _next_a():
            pltpu.make_async_copy(toka, out_slice(c0), ssem_a).wait()
            start_gather(c0 + 2, toka, cba, gsem_a)

        wait_gather(tokb, cbb, gsem_b)
        fma(tokb, cbb)
        pltpu.make_async_copy(tokb, out_slice(c0 + 1), ssem_b).start()

    pltpu.make_async_copy(toka, out_slice(n_chunks - 2), ssem_a).wait()
    pltpu.make_async_copy(tokb, out_slice(n_chunks - 1), ssem_b).wait()


@functools.partial(jax.jit, static_argnames=("n_rows",))
def _sc_embed(token_table, x_idx, seg_idx, comb, n_rows):
    scale = float(math.sqrt(D))
    mesh = plsc.VectorSubcoreMesh(core_axis_name="c", subcore_axis_name="s")
    idx_rows = n_rows // NW // IDX_BLK
    grid_kernel = pl.kernel(
        functools.partial(_sc_body, scale, n_rows),
        out_type=jax.ShapeDtypeStruct((n_rows, D), jnp.float32),
        mesh=mesh,
        compiler_params=pltpu.CompilerParams(use_tc_tiling_on_sc=False),
        scratch_types=[
            pltpu.VMEM((idx_rows, IDX_BLK), jnp.int32),
            pltpu.VMEM((idx_rows, IDX_BLK), jnp.int32),
            pltpu.VMEM((CH, D), jnp.float32),
            pltpu.VMEM((CH, D), jnp.float32),
            pltpu.VMEM((CH, D), jnp.float32),
            pltpu.VMEM((CH, D), jnp.float32),
            pltpu.SemaphoreType.DMA,
            pltpu.SemaphoreType.DMA,
            pltpu.SemaphoreType.DMA,
            pltpu.SemaphoreType.DMA,
        ],
    )
    return grid_kernel(token_table, x_idx, seg_idx, comb)


def kernel(x, segment_info, token_table, pos_embedding, segment_table):
    B, S_in = x.shape
    n_rows = B * S_in
    assert S_in == S and n_rows % (NW * CH) == 0 and CH % IDX_BLK == 0
    x_idx = x.reshape(n_rows // IDX_BLK, IDX_BLK).astype(jnp.int32)
    seg_idx = segment_info.reshape(n_rows // IDX_BLK, IDX_BLK).astype(jnp.int32)
    comb = (pos_embedding[0, :S_in, :][None, :, :]
            + segment_table[:, None, :]).reshape(-1, D)
    out = _sc_embed(token_table, x_idx, seg_idx, comb, n_rows)
    return out.reshape(B, S_in, D)


# speedup vs baseline: 1.5450x; 1.3778x over previous
import functools
import math

import jax
import jax.numpy as jnp
from jax import lax
from jax.experimental import pallas as pl
from jax.experimental.pallas import tpu as pltpu
from jax.experimental.pallas import tpu_sc as plsc

D = 64
LANES = 16
CH = 512
IDX_BLK = 128
NC = 2
NS = 16
NW = NC * NS
S = 512


def _sc_body(scale, n_rows, tok_hbm, x_hbm, seg_hbm, comb_hbm, out_hbm,
             xidx_all, p2, d1, toka, tokb, sega, segb,
             gsem_a, gsem_b, ssem_a, ssem_b):
    wid = lax.axis_index("s") * NC + lax.axis_index("c")
    rows_per_w = n_rows // NW
    n_chunks = rows_per_w // CH
    n_half = n_chunks // 2
    idx_rows = rows_per_w // IDX_BLK
    blk = CH // IDX_BLK

    pltpu.sync_copy(x_hbm.at[pl.ds(wid * idx_rows, idx_rows)], xidx_all)
    pltpu.sync_copy(comb_hbm.at[pl.ds(0, S)], p2)
    pltpu.sync_copy(comb_hbm.at[pl.ds(S, 1)], d1)
    delta = [d1[0, pl.ds(k * LANES, LANES)] - p2[0, pl.ds(k * LANES, LANES)]
             for k in range(D // LANES)]

    def start_gather(c, tokbuf, segbuf, gsem):
        for j in range(blk):
            pltpu.make_async_copy(
                tok_hbm.at[xidx_all.at[blk * c + j]],
                tokbuf.at[pl.ds(j * IDX_BLK, IDX_BLK)], gsem).start()
        pltpu.make_async_copy(
            seg_hbm.at[pl.ds(wid * rows_per_w + c * CH, CH)], segbuf,
            gsem).start()

    def wait_gather(tokbuf, segbuf, gsem):
        pltpu.make_async_copy(tok_hbm.at[pl.ds(0, CH)], tokbuf, gsem).wait()
        pltpu.make_async_copy(seg_hbm.at[pl.ds(0, CH)], segbuf, gsem).wait()

    def fma(tokbuf, segbuf):
        @pl.loop(0, CH // LANES)
        def _grp(g):
            r0 = g * LANES
            sv = segbuf[pl.ds(r0, LANES)].astype(jnp.float32)
            for i in range(LANES):
                r = r0 + i
                sf = sv[i]
                for k in range(D // LANES):
                    sl = pl.ds(k * LANES, LANES)
                    tokbuf[r, sl] = (tokbuf[r, sl] * scale
                                     + (p2[r, sl] + sf * delta[k]))

    def out_slice(c):
        return out_hbm.at[pl.ds(wid * rows_per_w + c * CH, CH)]

    start_gather(0, toka, sega, gsem_a)

    @pl.loop(0, n_half)
    def _pipe(t):
        c0 = 2 * t

        @pl.when(t > 0)
        def _free_b():
            pltpu.make_async_copy(tokb, out_slice(c0 - 1), ssem_b).wait()

        start_gather(c0 + 1, tokb, segb, gsem_b)

        wait_gather(toka, sega, gsem_a)
        fma(toka, sega)
        pltpu.make_async_copy(toka, out_slice(c0), ssem_a).start()

        @pl.when(t + 1 < n_half)
        def _next_a():
            pltpu.make_async_copy(toka, out_slice(c0), ssem_a).wait()
            start_gather(c0 + 2, toka, sega, gsem_a)

        wait_gather(tokb, segb, gsem_b)
        fma(tokb, segb)
        pltpu.make_async_copy(tokb, out_slice(c0 + 1), ssem_b).start()

    pltpu.make_async_copy(toka, out_slice(n_chunks - 2), ssem_a).wait()
    pltpu.make_async_copy(tokb, out_slice(n_chunks - 1), ssem_b).wait()


@functools.partial(jax.jit, static_argnames=("n_rows",))
def _sc_embed(token_table, x_idx, seg_flat, comb, n_rows):
    scale = float(math.sqrt(D))
    mesh = plsc.VectorSubcoreMesh(core_axis_name="c", subcore_axis_name="s")
    idx_rows = n_rows // NW // IDX_BLK
    grid_kernel = pl.kernel(
        functools.partial(_sc_body, scale, n_rows),
        out_type=jax.ShapeDtypeStruct((n_rows, D), jnp.float32),
        mesh=mesh,
        compiler_params=pltpu.CompilerParams(use_tc_tiling_on_sc=False),
        scratch_types=[
            pltpu.VMEM((idx_rows, IDX_BLK), jnp.int32),
            pltpu.VMEM((S, D), jnp.float32),
            pltpu.VMEM((1, D), jnp.float32),
            pltpu.VMEM((CH, D), jnp.float32),
            pltpu.VMEM((CH, D), jnp.float32),
            pltpu.VMEM((CH,), jnp.int32),
            pltpu.VMEM((CH,), jnp.int32),
            pltpu.SemaphoreType.DMA,
            pltpu.SemaphoreType.DMA,
            pltpu.SemaphoreType.DMA,
            pltpu.SemaphoreType.DMA,
        ],
    )
    return grid_kernel(token_table, x_idx, seg_flat, comb)


def kernel(x, segment_info, token_table, pos_embedding, segment_table):
    B, S_in = x.shape
    n_rows = B * S_in
    assert S_in == S and n_rows % (NW * CH) == 0 and CH % IDX_BLK == 0
    x_idx = x.reshape(n_rows // IDX_BLK, IDX_BLK).astype(jnp.int32)
    seg_flat = segment_info.reshape(n_rows).astype(jnp.int32)
    comb = (pos_embedding[0, :S_in, :][None, :, :]
            + segment_table[:, None, :]).reshape(-1, D)
    out = _sc_embed(token_table, x_idx, seg_flat, comb, n_rows)
    return out.reshape(B, S_in, D)
